# Initial kernel scaffold; baseline (speedup 1.0000x reference)
import jax
import jax.numpy as jnp
from jax.experimental import pallas as pl


def kernel(xyz):
    raise NotImplementedError("write your pallas kernel here")


# speedup vs baseline: 15.6506x; 15.6506x over previous
import functools

import jax
import jax.numpy as jnp
from jax.experimental import pallas as pl

_NQ = 1024
_K = 16
_QBLK = 256


def _knn_kernel(q_ref, x_ref, out_ref, *, n):
    q = q_ref[0]
    x0 = x_ref[0, 0:1, :]
    x1 = x_ref[0, 1:2, :]
    x2 = x_ref[0, 2:3, :]
    d0 = q[:, 0:1] - x0
    d1 = q[:, 1:2] - x1
    d2c = q[:, 2:3] - x2
    d2 = (d0 * d0 + d1 * d1) + d2c * d2c

    iota = jax.lax.broadcasted_iota(jnp.int32, d2.shape, 1)
    big = jnp.float32(jnp.inf)
    cols = []
    for _ in range(_K):
        m = jnp.min(d2, axis=1, keepdims=True)
        cand = jnp.where(d2 == m, iota, jnp.int32(n))
        idx = jnp.min(cand, axis=1, keepdims=True)
        cols.append(idx)
        d2 = jnp.where(iota == idx, big, d2)
    out_ref[0] = jnp.concatenate(cols, axis=1)


def kernel(xyz):
    b, n, _ = xyz.shape
    stride = n // _NQ
    queries = xyz[:, ::stride, :]
    xt = jnp.transpose(xyz, (0, 2, 1))

    knn_idx = pl.pallas_call(
        functools.partial(_knn_kernel, n=n),
        grid=(b, _NQ // _QBLK),
        in_specs=[
            pl.BlockSpec((1, _QBLK, 3), lambda i, j: (i, j, 0)),
            pl.BlockSpec((1, 3, n), lambda i, j: (i, 0, 0)),
        ],
        out_specs=pl.BlockSpec((1, _QBLK, _K), lambda i, j: (i, j, 0)),
        out_shape=jax.ShapeDtypeStruct((b, _NQ, _K), jnp.int32),
    )(queries, xt)

    out_indices = knn_idx.astype(jnp.int64)[..., None]
    return out_indices, queries


# speedup vs baseline: 16.0050x; 1.0227x over previous
import functools

import jax
import jax.numpy as jnp
from jax.experimental import pallas as pl

_NQ = 1024
_K = 16
_QBLK = 256


def _knn_kernel(q_ref, x_ref, out_ref, *, n):
    q = q_ref[0]
    x0 = x_ref[0, 0:1, :]
    x1 = x_ref[0, 1:2, :]
    x2 = x_ref[0, 2:3, :]
    d0 = q[:, 0:1] - x0
    d1 = q[:, 1:2] - x1
    d2c = q[:, 2:3] - x2
    d2 = (d0 * d0 + d1 * d1) + d2c * d2c

    iota = jax.lax.broadcasted_iota(jnp.int32, d2.shape, 1)
    big = jnp.float32(jnp.inf)
    cols = []
    for _ in range(_K):
        idx = jnp.argmin(d2, axis=1)[:, None]
        cols.append(idx.astype(jnp.int32))
        d2 = jnp.where(iota == idx, big, d2)
    out_ref[0] = jnp.concatenate(cols, axis=1)


def kernel(xyz):
    b, n, _ = xyz.shape
    stride = n // _NQ
    queries = xyz[:, ::stride, :]
    xt = jnp.transpose(xyz, (0, 2, 1))

    knn_idx = pl.pallas_call(
        functools.partial(_knn_kernel, n=n),
        grid=(b, _NQ // _QBLK),
        in_specs=[
            pl.BlockSpec((1, _QBLK, 3), lambda i, j: (i, j, 0)),
            pl.BlockSpec((1, 3, n), lambda i, j: (i, 0, 0)),
        ],
        out_specs=pl.BlockSpec((1, _QBLK, _K), lambda i, j: (i, j, 0)),
        out_shape=jax.ShapeDtypeStruct((b, _NQ, _K), jnp.int32),
    )(queries, xt)

    out_indices = knn_idx.astype(jnp.int64)[..., None]
    return out_indices, queries
